# baseline (speedup 1.0000x reference)
import functools

import jax
import jax.numpy as jnp
from jax import lax
from jax.experimental import pallas as pl
from jax.experimental.pallas import tpu as pltpu
from jax.experimental.pallas import tpu_sc as plsc

E = 16
BT = 1024
TPW = 512
NW = 32
NEG_INF = float("-inf")


def _logits_body(x_ref, gw_ref, o_ref):
    lT = jax.lax.dot_general(
        gw_ref[...], x_ref[...].astype(jnp.bfloat16),
        (((1,), (1,)), ((), ())),
        preferred_element_type=jnp.float32)
    o_ref[...] = lT.reshape(E, BT // TPW, TPW).transpose(1, 0, 2)


def _routing_body(lT_hbm, w_hbm, lv, wv):
    wid = lax.axis_index("s") * 2 + lax.axis_index("c")
    pltpu.sync_copy(lT_hbm.at[wid], lv)

    def chunk(j, carry):
        base = j * 16
        ls = [lv[e, pl.ds(base, 16)] for e in range(E)]
        v0 = ls[0]
        for e in range(1, E):
            v0 = jnp.maximum(v0, ls[e])
        m0 = [l == v0 for l in ls]
        rest = [jnp.where(m0[e], NEG_INF, ls[e]) for e in range(E)]
        v1 = rest[0]
        for e in range(1, E):
            v1 = jnp.maximum(v1, rest[e])
        m1 = [r == v1 for r in rest]
        w0 = 1.0 / (1.0 + jnp.exp(v1 - v0))
        w1 = 1.0 - w0
        zero = jnp.zeros((16,), jnp.float32)
        for e in range(E):
            wv[e, pl.ds(base, 16)] = jnp.where(
                m0[e], w0, jnp.where(m1[e], w1, zero))
        return carry

    lax.fori_loop(0, TPW // 16, chunk, 0)
    pltpu.sync_copy(wv, w_hbm.at[wid])


def _combine_body(x_ref, wT_ref, ewm1_ref, o_ref):
    x = x_ref[...]
    wT = wT_ref[...].transpose(1, 0, 2).reshape(E, BT)
    corr = jax.lax.dot_general(
        wT.astype(jnp.bfloat16), ewm1_ref[...],
        (((0,), (0,)), ((), ())),
        preferred_element_type=jnp.float32)
    o_ref[...] = x + x * corr


@functools.partial(jax.jit, static_argnames=())
def kernel(norm_data, gate_w, expert_w):
    T, D = norm_data.shape
    gw_b = gate_w.astype(jnp.bfloat16)
    ewm1_b = (expert_w - 1.0).astype(jnp.bfloat16)

    lT = pl.pallas_call(
        _logits_body,
        grid=(T // BT,),
        in_specs=[
            pl.BlockSpec((BT, D), lambda i: (i, 0)),
            pl.BlockSpec((E, D), lambda i: (0, 0)),
        ],
        out_specs=pl.BlockSpec((BT // TPW, E, TPW), lambda i: (i, 0, 0)),
        out_shape=jax.ShapeDtypeStruct((T // TPW, E, TPW), jnp.float32),
        compiler_params=pltpu.CompilerParams(
            dimension_semantics=("arbitrary",),
        ),
    )(norm_data, gw_b)

    mesh = plsc.VectorSubcoreMesh(core_axis_name="c", subcore_axis_name="s")
    routing = pl.kernel(
        _routing_body,
        mesh=mesh,
        out_type=jax.ShapeDtypeStruct((NW, E, TPW), jnp.float32),
        scratch_types=[
            pltpu.VMEM((E, TPW), jnp.float32),
            pltpu.VMEM((E, TPW), jnp.float32),
        ],
    )
    wT = routing(lT)

    return pl.pallas_call(
        _combine_body,
        grid=(T // BT,),
        in_specs=[
            pl.BlockSpec((BT, D), lambda i: (i, 0)),
            pl.BlockSpec((BT // TPW, E, TPW), lambda i: (i, 0, 0)),
            pl.BlockSpec((E, D), lambda i: (0, 0)),
        ],
        out_specs=pl.BlockSpec((BT, D), lambda i: (i, 0)),
        out_shape=jax.ShapeDtypeStruct((T, D), norm_data.dtype),
        compiler_params=pltpu.CompilerParams(
            dimension_semantics=("arbitrary",),
        ),
    )(norm_data, wT, ewm1_b)


# speedup vs baseline: 1.5464x; 1.5464x over previous
import functools

import jax
import jax.numpy as jnp
from jax.experimental import pallas as pl
from jax.experimental.pallas import tpu as pltpu

E = 16
TOPK = 2
BLOCK_T = 1664


def _moe_body(x_ref, gw_ref, ewm1_ref, o_ref):
    x = x_ref[...]
    logits = jax.lax.dot_general(
        x, gw_ref[...].astype(jnp.float32), (((1,), (1,)), ((), ())),
        preferred_element_type=jnp.float32)

    v0 = jnp.max(logits, axis=1, keepdims=True)
    mask0 = logits == v0
    rest = jnp.where(mask0, -jnp.inf, logits)
    v1 = jnp.max(rest, axis=1, keepdims=True)
    mask1 = rest == v1

    w0 = 1.0 / (1.0 + jnp.exp(v1 - v0))
    w = jnp.where(mask0, w0, 0.0) + jnp.where(mask1, 1.0 - w0, 0.0)

    corr = jax.lax.dot_general(
        w.astype(jnp.bfloat16), ewm1_ref[...], (((1,), (0,)), ((), ())),
        preferred_element_type=jnp.float32).astype(jnp.bfloat16)
    o_ref[...] = x + x * corr.astype(jnp.float32)


@functools.partial(jax.jit, static_argnames=())
def kernel(norm_data, gate_w, expert_w):
    T, D = norm_data.shape
    gw_b = gate_w.astype(jnp.bfloat16)
    ewm1_b = (expert_w - 1.0).astype(jnp.bfloat16)
    grid = (pl.cdiv(T, BLOCK_T),)
    return pl.pallas_call(
        _moe_body,
        grid=grid,
        in_specs=[
            pl.BlockSpec((BLOCK_T, D), lambda i: (i, 0)),
            pl.BlockSpec((E, D), lambda i: (0, 0)),
            pl.BlockSpec((E, D), lambda i: (0, 0)),
        ],
        out_specs=pl.BlockSpec((BLOCK_T, D), lambda i: (i, 0)),
        out_shape=jax.ShapeDtypeStruct((T, D), norm_data.dtype),
        compiler_params=pltpu.CompilerParams(
            dimension_semantics=("parallel",),
        ),
    )(norm_data, gw_b, ewm1_b)
